# baseline (speedup 1.0000x reference)
import functools

import jax
import jax.numpy as jnp
from jax import lax
from jax.experimental import pallas as pl
from jax.experimental.pallas import tpu as pltpu
from jax.experimental.pallas import tpu_sc as plsc

_NC, _NS, _LANES = 2, 16, 16
_NW = _NC * _NS


def _sc_gather_combine(idx_flat, concept_table, law_table, a16, b16):
    N = idx_flat.shape[0]
    D = concept_table.shape[1]
    per_w = N // _NW
    C = 128
    n_chunks = per_w // C

    mesh = plsc.VectorSubcoreMesh(core_axis_name="c", subcore_axis_name="s")

    @functools.partial(
        pl.kernel,
        mesh=mesh,
        out_type=jax.ShapeDtypeStruct((N, D), jnp.float32),
        scratch_types=[
            pltpu.VMEM((C,), jnp.int32),
            pltpu.VMEM((C, D), jnp.float32),
            pltpu.VMEM((C, D), jnp.float32),
            pltpu.VMEM((_LANES,), jnp.float32),
            pltpu.VMEM((_LANES,), jnp.float32),
            pltpu.SemaphoreType.DMA,
        ],
        compiler_params=pltpu.CompilerParams(use_tc_tiling_on_sc=False),
    )
    def k(idx_hbm, ct_hbm, lt_hbm, a_hbm, b_hbm, out_hbm, idx_v, c_v, l_v, a_v, b_v, sem):
        wid = lax.axis_index("s") * _NC + lax.axis_index("c")
        base = wid * per_w
        pltpu.sync_copy(a_hbm, a_v)
        pltpu.sync_copy(b_hbm, b_v)
        av = a_v[...]
        bv = b_v[...]

        def chunk(g, carry):
            off = base + g * C
            pltpu.sync_copy(idx_hbm.at[pl.ds(off, C)], idx_v)
            cp1 = pltpu.async_copy(ct_hbm.at[idx_v], c_v, sem)
            cp2 = pltpu.async_copy(lt_hbm.at[idx_v], l_v, sem)
            cp1.wait()
            cp2.wait()

            def row(r, carry2):
                for kk in range(D // _LANES):
                    s = pl.ds(kk * _LANES, _LANES)
                    c_v[r, s] = av * c_v[r, s] + bv * l_v[r, s]
                return carry2

            lax.fori_loop(0, C, row, 0)
            pltpu.sync_copy(c_v, out_hbm.at[pl.ds(off, C)])
            return carry

        lax.fori_loop(0, n_chunks, chunk, 0)

    return k(idx_flat, concept_table, law_table, a16, b16)


def _tc_body(x_ref, comb_ref, g_ref, lb_ref, w1_ref, b1_ref, w2_ref, b2_ref, out_ref):
    xp = x_ref[...] + comb_ref[...]
    mu = jnp.mean(xp, axis=1, keepdims=True)
    xc = xp - mu
    var = jnp.mean(xc * xc, axis=1, keepdims=True)
    h = xc * lax.rsqrt(var + 1e-5) * g_ref[...] + lb_ref[...]
    hid = jnp.dot(h, w1_ref[...], preferred_element_type=jnp.float32) + b1_ref[...]
    act = 0.5 * hid * (1.0 + lax.erf(hid * 0.7071067811865476))
    ffn = jnp.dot(act, w2_ref[...], preferred_element_type=jnp.float32) + b2_ref[...]
    out_ref[...] = xp + ffn


def _tc_ffn(xf, comb, ln_gamma, ln_beta, W1, b1, W2, b2):
    N, D = xf.shape
    H = W1.shape[1]
    TN = 1024
    grid = (N // TN,)
    return pl.pallas_call(
        _tc_body,
        grid=grid,
        in_specs=[
            pl.BlockSpec((TN, D), lambda i: (i, 0)),
            pl.BlockSpec((TN, D), lambda i: (i, 0)),
            pl.BlockSpec((1, D), lambda i: (0, 0)),
            pl.BlockSpec((1, D), lambda i: (0, 0)),
            pl.BlockSpec((D, H), lambda i: (0, 0)),
            pl.BlockSpec((1, H), lambda i: (0, 0)),
            pl.BlockSpec((H, D), lambda i: (0, 0)),
            pl.BlockSpec((1, D), lambda i: (0, 0)),
        ],
        out_specs=pl.BlockSpec((TN, D), lambda i: (i, 0)),
        out_shape=jax.ShapeDtypeStruct((N, D), jnp.float32),
    )(xf, comb, ln_gamma.reshape(1, D), ln_beta.reshape(1, D),
      W1, b1.reshape(1, H), W2, b2.reshape(1, D))


def kernel(x, idx, concept_table, law_table, alpha, beta, ln_gamma, ln_beta, W1, b1, W2, b2):
    B, L, D = x.shape
    N = B * L
    xf = x.reshape(N, D)
    idx_flat = idx.reshape(N).astype(jnp.int32)
    a16 = jnp.full((_LANES,), alpha, jnp.float32)
    b16 = jnp.full((_LANES,), beta, jnp.float32)
    comb = _sc_gather_combine(idx_flat, concept_table, law_table, a16, b16)
    out = _tc_ffn(xf, comb, ln_gamma, ln_beta, W1, b1, W2, b2)
    return out.reshape(B, L, D)
